# baseline (speedup 1.0000x reference)
import functools

import jax
import jax.numpy as jnp
from jax import lax
from jax.experimental import pallas as pl
from jax.experimental.pallas import tpu as pltpu
from jax.experimental.pallas import tpu_sc as plsc

VOCAB = 1000000
EMB = 64

_info = plsc.get_sparse_core_info()
NC, NS, L = _info.num_cores, _info.num_subcores, _info.num_lanes
NW = NC * NS

B_TOTAL = 4096 * 200
PER_W = B_TOTAL // NW
V_ROWS = PER_W // 128
CHUNK = 128
N_CHUNKS = PER_W // CHUNK
NBUF = 2


def _body(idx_hbm, tab_hbm, out_hbm, idx_v, g, o, gsems, ssems):
    c = lax.axis_index("c")
    s = lax.axis_index("s")
    wid = s * NC + c
    base = wid * PER_W

    pltpu.sync_copy(idx_hbm.at[wid], idx_v)

    def fire_gather(ci, b):
        pltpu.async_copy(tab_hbm.at[idx_v.at[ci]], g[b], gsems[b])

    def wait_gather(b):
        pltpu.make_async_copy(tab_hbm.at[pl.ds(0, CHUNK)], g[b], gsems[b]).wait()

    def compact(b):
        def row_body(r16, _):
            for u in range(L):
                r = r16 * L + u
                for cg in range(EMB // L):
                    o[b][r, pl.ds(cg * L, L)] = g[b][r, pl.ds(cg * L, L)]
            return 0

        lax.fori_loop(0, CHUNK // L, row_body, 0)

    def fire_store(ci, b):
        pltpu.async_copy(
            o[b], out_hbm.at[pl.ds(base + ci * CHUNK, CHUNK)], ssems[b]
        )

    def wait_store(b):
        pltpu.make_async_copy(
            o[b], out_hbm.at[pl.ds(base, CHUNK)], ssems[b]
        ).wait()

    fire_gather(0, 0)

    def group(gidx, _):
        for b in range(NBUF):
            ci = gidx * NBUF + b
            wait_gather(b)
            nb = 1 - b
            @pl.when(ci + 1 < N_CHUNKS)
            def _():
                fire_gather(ci + 1, nb)
            @pl.when(ci >= 2)
            def _():
                wait_store(b)
            compact(b)
            fire_store(ci, b)
        return 0

    lax.fori_loop(0, N_CHUNKS // NBUF, group, 0)
    for b in range(NBUF):
        wait_store(b)


FT = VOCAB // 128
WT_BASE = FT // NW
WT_REM = FT - WT_BASE * NW


def _transpose_body(tabT_hbm, tab128_hbm, tin, tout, rsems, wsems):
    c = lax.axis_index("c")
    s = lax.axis_index("s")
    wid = s * NC + c
    t0 = wid * WT_BASE + jnp.minimum(wid, WT_REM)
    nt = WT_BASE + jnp.where(wid < WT_REM, 1, 0)
    NT = WT_BASE + 1

    def tile_of(ti):
        return t0 + jnp.minimum(ti, nt - 1)

    def fire_reads(ti, b):
        for eT in range(8):
            pltpu.async_copy(
                tabT_hbm.at[pl.ds(eT * 8, 8), pl.ds(tile_of(ti) * 128, 128)],
                tin[b].at[pl.ds(eT * 8, 8)],
                rsems[b],
            )

    def wait_reads(b):
        pltpu.make_async_copy(
            tabT_hbm.at[pl.ds(0, 64), pl.ds(0, 128)], tin[b], rsems[b]
        ).wait()

    def transpose(b):
        iota = lax.iota(jnp.int32, L)
        for e in range(EMB):
            cole = jnp.full((L,), e, jnp.int32)
            for cg in range(128 // L):
                vals = tin[b][e, pl.ds(cg * L, L)]
                plsc.store_scatter(tout[b], [iota + cg * L, cole], vals)

    def fire_write(ti, b):
        pltpu.async_copy(
            tout[b], tab128_hbm.at[pl.ds(tile_of(ti) * 128, 128)], wsems[b]
        )

    def wait_write(b):
        pltpu.make_async_copy(
            tout[b], tab128_hbm.at[pl.ds(0, 128)], wsems[b]
        ).wait()

    fire_reads(0, 0)

    def group(gi, _):
        for b in range(2):
            ti = gi * 2 + b
            nb = 1 - b

            @pl.when(ti < NT)
            def _():
                @pl.when(ti + 1 < NT)
                def _():
                    fire_reads(ti + 1, nb)
                wait_reads(b)
                @pl.when(ti >= 2)
                def _():
                    wait_write(b)
                transpose(b)
                fire_write(ti, b)
        return 0

    lax.fori_loop(0, (NT + 1) // 2, group, 0)
    for b in range(2):
        wait_write(b)


@jax.jit
def _build_tab128(tabT):
    mesh = plsc.VectorSubcoreMesh(core_axis_name="c", subcore_axis_name="s")
    f = pl.kernel(
        _transpose_body,
        out_type=jax.ShapeDtypeStruct((VOCAB, 128), jnp.float32),
        mesh=mesh,
        scratch_types=[
            [pltpu.VMEM((64, 128), jnp.float32) for _ in range(2)],
            [pltpu.VMEM((128, 128), jnp.float32) for _ in range(2)],
            [pltpu.SemaphoreType.DMA for _ in range(2)],
            [pltpu.SemaphoreType.DMA for _ in range(2)],
        ],
        compiler_params=pltpu.CompilerParams(
            use_tc_tiling_on_sc=True, needs_layout_passes=False
        ),
    )
    return f(tabT)


@jax.jit
def _gather(idx, tab128):
    mesh = plsc.VectorSubcoreMesh(core_axis_name="c", subcore_axis_name="s")
    f = pl.kernel(
        _body,
        out_type=jax.ShapeDtypeStruct((B_TOTAL, EMB), jnp.float32),
        mesh=mesh,
        scratch_types=[
            pltpu.VMEM((V_ROWS, 128), jnp.int32),
            [pltpu.VMEM((CHUNK, 128), jnp.float32) for _ in range(NBUF)],
            [pltpu.VMEM((CHUNK, EMB), jnp.float32) for _ in range(NBUF)],
            [pltpu.SemaphoreType.DMA for _ in range(NBUF)],
            [pltpu.SemaphoreType.DMA for _ in range(NBUF)],
        ],
        compiler_params=pltpu.CompilerParams(use_tc_tiling_on_sc=True),
    )
    return f(idx, tab128)


def kernel(data, tvectors):
    idx = data.astype(jnp.int32).reshape(NW, V_ROWS, 128)
    tab128 = _build_tab128(tvectors.T)
    tail = jnp.pad(tvectors[FT * 128 :], ((0, 0), (0, EMB)))
    tab128 = jax.lax.dynamic_update_slice(tab128, tail, (FT * 128, 0))
    out = _gather(idx, tab128)
    return out.reshape(data.shape[0], data.shape[1], EMB)


# speedup vs baseline: 1.6682x; 1.6682x over previous
import functools

import jax
import jax.numpy as jnp
from jax import lax
from jax.experimental import pallas as pl
from jax.experimental.pallas import tpu as pltpu
from jax.experimental.pallas import tpu_sc as plsc

VOCAB = 1000000
EMB = 64

_info = plsc.get_sparse_core_info()
NC, NS, L = _info.num_cores, _info.num_subcores, _info.num_lanes
NW = NC * NS

B_TOTAL = 4096 * 200
PER_W = B_TOTAL // NW
V_ROWS = PER_W // 128
CHUNK = 128
N_CHUNKS = PER_W // CHUNK
NBUF = 2


def _body(idx_hbm, tab_hbm, out_hbm, idx_v, g, o, gsems, ssems):
    c = lax.axis_index("c")
    s = lax.axis_index("s")
    wid = s * NC + c
    base = wid * PER_W

    pltpu.sync_copy(idx_hbm.at[wid], idx_v)

    def fire_gather(ci, b):
        pltpu.async_copy(tab_hbm.at[idx_v.at[ci]], g[b], gsems[b])

    def wait_gather(b):
        pltpu.make_async_copy(tab_hbm.at[pl.ds(0, CHUNK)], g[b], gsems[b]).wait()

    def compact(b):
        def row_body(r16, _):
            for u in range(L):
                r = r16 * L + u
                for cg in range(EMB // L):
                    o[b][r, pl.ds(cg * L, L)] = g[b][r, pl.ds(cg * L, L)]
            return 0

        lax.fori_loop(0, CHUNK // L, row_body, 0)

    def fire_store(ci, b):
        pltpu.async_copy(
            o[b], out_hbm.at[pl.ds(base + ci * CHUNK, CHUNK)], ssems[b]
        )

    def wait_store(b):
        pltpu.make_async_copy(
            o[b], out_hbm.at[pl.ds(base, CHUNK)], ssems[b]
        ).wait()

    fire_gather(0, 0)

    def group(gidx, _):
        for b in range(NBUF):
            ci = gidx * NBUF + b
            wait_gather(b)
            nb = 1 - b
            @pl.when(ci + 1 < N_CHUNKS)
            def _():
                fire_gather(ci + 1, nb)
            @pl.when(ci >= 2)
            def _():
                wait_store(b)
            compact(b)
            fire_store(ci, b)
        return 0

    lax.fori_loop(0, N_CHUNKS // NBUF, group, 0)
    for b in range(NBUF):
        wait_store(b)


@jax.jit
def _gather(idx, tab128):
    mesh = plsc.VectorSubcoreMesh(core_axis_name="c", subcore_axis_name="s")
    f = pl.kernel(
        _body,
        out_type=jax.ShapeDtypeStruct((B_TOTAL, EMB), jnp.float32),
        mesh=mesh,
        scratch_types=[
            pltpu.VMEM((V_ROWS, 128), jnp.int32),
            [pltpu.VMEM((CHUNK, 128), jnp.float32) for _ in range(NBUF)],
            [pltpu.VMEM((CHUNK, EMB), jnp.float32) for _ in range(NBUF)],
            [pltpu.SemaphoreType.DMA for _ in range(NBUF)],
            [pltpu.SemaphoreType.DMA for _ in range(NBUF)],
        ],
        compiler_params=pltpu.CompilerParams(use_tc_tiling_on_sc=True),
    )
    return f(idx, tab128)


def kernel(data, tvectors):
    idx = data.astype(jnp.int32).reshape(NW, V_ROWS, 128)
    tab128 = jnp.pad(tvectors, ((0, 0), (0, EMB)))
    out = _gather(idx, tab128)
    return out.reshape(data.shape[0], data.shape[1], EMB)


# speedup vs baseline: 1.7569x; 1.0531x over previous
import functools

import jax
import jax.numpy as jnp
from jax import lax
from jax.experimental import pallas as pl
from jax.experimental.pallas import tpu as pltpu
from jax.experimental.pallas import tpu_sc as plsc

VOCAB = 1000000
EMB = 64

_info = plsc.get_sparse_core_info()
NC, NS, L = _info.num_cores, _info.num_subcores, _info.num_lanes
NW = NC * NS

B_TOTAL = 4096 * 200
PER_W = B_TOTAL // NW
V_ROWS = PER_W // 128
CHUNK = 128
N_CHUNKS = PER_W // CHUNK
NBUF = 4


def _body(idx_hbm, tab_hbm, out_hbm, idx_v, g, o, gsems, ssems):
    c = lax.axis_index("c")
    s = lax.axis_index("s")
    wid = s * NC + c
    base = wid * PER_W

    pltpu.sync_copy(idx_hbm.at[wid], idx_v)

    def fire_gather(ci, b):
        pltpu.async_copy(tab_hbm.at[idx_v.at[ci]], g[b], gsems[b])

    def wait_gather(b):
        pltpu.make_async_copy(tab_hbm.at[pl.ds(0, CHUNK)], g[b], gsems[b]).wait()

    def compact(b, ob):
        def row_body(r16, _):
            for u in range(L):
                r = r16 * L + u
                for cg in range(EMB // L):
                    o[ob][r, pl.ds(cg * L, L)] = g[b][r, pl.ds(cg * L, L)]
            return 0

        lax.fori_loop(0, CHUNK // L, row_body, 0)

    def fire_store(ci, ob):
        pltpu.async_copy(
            o[ob], out_hbm.at[pl.ds(base + ci * CHUNK, CHUNK)], ssems[ob]
        )

    def wait_store(ob):
        pltpu.make_async_copy(
            o[ob], out_hbm.at[pl.ds(base, CHUNK)], ssems[ob]
        ).wait()

    for p in range(NBUF - 1):
        fire_gather(p, p)

    def group(gidx, _):
        for b in range(NBUF):
            ci = gidx * NBUF + b
            ob = b % 2
            wait_gather(b)
            @pl.when(ci + NBUF - 1 < N_CHUNKS)
            def _():
                fire_gather(ci + NBUF - 1, (b + NBUF - 1) % NBUF)
            @pl.when(ci >= 2)
            def _():
                wait_store(ob)
            compact(b, ob)
            fire_store(ci, ob)
        return 0

    lax.fori_loop(0, N_CHUNKS // NBUF, group, 0)
    for ob in range(2):
        wait_store(ob)


@jax.jit
def _gather(idx, tab128):
    mesh = plsc.VectorSubcoreMesh(core_axis_name="c", subcore_axis_name="s")
    f = pl.kernel(
        _body,
        out_type=jax.ShapeDtypeStruct((B_TOTAL, EMB), jnp.float32),
        mesh=mesh,
        scratch_types=[
            pltpu.VMEM((V_ROWS, 128), jnp.int32),
            [pltpu.VMEM((CHUNK, 128), jnp.float32) for _ in range(NBUF)],
            [pltpu.VMEM((CHUNK, EMB), jnp.float32) for _ in range(2)],
            [pltpu.SemaphoreType.DMA for _ in range(NBUF)],
            [pltpu.SemaphoreType.DMA for _ in range(2)],
        ],
        compiler_params=pltpu.CompilerParams(use_tc_tiling_on_sc=True),
    )
    return f(idx, tab128)


def kernel(data, tvectors):
    idx = data.astype(jnp.int32).reshape(NW, V_ROWS, 128)
    tab128 = jnp.pad(tvectors, ((0, 0), (0, EMB)))
    out = _gather(idx, tab128)
    return out.reshape(data.shape[0], data.shape[1], EMB)


# speedup vs baseline: 1.7584x; 1.0009x over previous
import functools

import jax
import jax.numpy as jnp
from jax import lax
from jax.experimental import pallas as pl
from jax.experimental.pallas import tpu as pltpu
from jax.experimental.pallas import tpu_sc as plsc

VOCAB = 1000000
EMB = 64

_info = plsc.get_sparse_core_info()
NC, NS, L = _info.num_cores, _info.num_subcores, _info.num_lanes
NW = NC * NS

B_TOTAL = 4096 * 200
PER_W = B_TOTAL // NW
V_ROWS = PER_W // 128
CHUNK = 128
N_CHUNKS = PER_W // CHUNK
NBUF = 4


def _body(idx_hbm, tab_hbm, out_hbm, idx_v, g, o, gsems, ssems):
    c = lax.axis_index("c")
    s = lax.axis_index("s")
    wid = s * NC + c
    base = wid * PER_W

    pltpu.sync_copy(idx_hbm.at[wid], idx_v)

    def fire_gather(ci, b):
        pltpu.async_copy(tab_hbm.at[idx_v.at[ci]], g[b], gsems[b])

    def wait_gather(b):
        pltpu.make_async_copy(tab_hbm.at[pl.ds(0, CHUNK)], g[b], gsems[b]).wait()

    def compact(b, ob):
        def row_body(r16, _):
            for u in range(L):
                r = r16 * L + u
                for cg in range(EMB // L):
                    o[ob][r, pl.ds(cg * L, L)] = g[b][r, pl.ds(cg * L, L)]
            return 0

        lax.fori_loop(0, CHUNK // L, row_body, 0)

    def fire_store(ci, ob):
        pltpu.async_copy(
            o[ob], out_hbm.at[pl.ds(base + ci * CHUNK, CHUNK)], ssems[ob]
        )

    def wait_store(ob):
        pltpu.make_async_copy(
            o[ob], out_hbm.at[pl.ds(base, CHUNK)], ssems[ob]
        ).wait()

    for p in range(NBUF - 1):
        fire_gather(p, p)

    def group(gidx, _):
        for b in range(NBUF):
            ci = gidx * NBUF + b
            ob = b % 2
            wait_gather(b)
            @pl.when(ci + NBUF - 1 < N_CHUNKS)
            def _():
                fire_gather(ci + NBUF - 1, (b + NBUF - 1) % NBUF)
            @pl.when(ci >= 2)
            def _():
                wait_store(ob)
            compact(b, ob)
            fire_store(ci, ob)
        return 0

    lax.fori_loop(0, N_CHUNKS // NBUF, group, 0)
    for ob in range(2):
        wait_store(ob)


@jax.jit
def _gather(idx, tab128):
    mesh = plsc.VectorSubcoreMesh(core_axis_name="c", subcore_axis_name="s")
    f = pl.kernel(
        _body,
        out_type=jax.ShapeDtypeStruct((B_TOTAL, EMB), jnp.float32),
        mesh=mesh,
        scratch_types=[
            pltpu.VMEM((V_ROWS, 128), jnp.int32),
            [pltpu.VMEM((CHUNK, 128), jnp.float32) for _ in range(NBUF)],
            [pltpu.VMEM((CHUNK, EMB), jnp.float32) for _ in range(2)],
            [pltpu.SemaphoreType.DMA for _ in range(NBUF)],
            [pltpu.SemaphoreType.DMA for _ in range(2)],
        ],
        compiler_params=pltpu.CompilerParams(use_tc_tiling_on_sc=True),
    )
    return f(idx, tab128)


def kernel(data, tvectors):
    idx = data.astype(jnp.int32).reshape(NW, V_ROWS, 128)
    tab128 = jnp.concatenate(
        [tvectors, jnp.zeros((VOCAB, EMB), jnp.float32)], axis=1
    )
    out = _gather(idx, tab128)
    return out.reshape(data.shape[0], data.shape[1], EMB)
